# baseline (speedup 1.0000x reference)
import functools

import jax
import jax.numpy as jnp
from jax import lax
from jax.experimental import pallas as pl
from jax.experimental.pallas import tpu as pltpu
from jax.experimental.pallas import tpu_sc as plsc

BATCH = 4096
NUM_FIELDS = 26
NC = 2
NS = 16
LANES = 16
NW = NC * NS
ROWS_PER_W = BATCH // NW
VOCAB_N = 100000
MAIN_N = 99328
TAIL_LEN = VOCAB_N - MAIN_N
SLICE = MAIN_N // NS
HALF = SLICE // 2
F_G0 = 13


def _sc_kernel(xt_hbm, tmain_hbm, ttail_hbm, bias_hbm, out_hbm, spt, idx_v,
               vals_v, out_v, bias_v, tab_v, tail_v, semi, semb, semta,
               semtb, semt2, semg0, semg1):
    cid = lax.axis_index("c")
    sid = lax.axis_index("s")
    wid = sid * NC + cid
    base = wid * ROWS_PER_W

    stage = [
        pltpu.async_copy(
            xt_hbm.at[f, pl.ds(base, ROWS_PER_W)],
            idx_v.at[pl.ds(f * ROWS_PER_W, ROWS_PER_W)],
            semi,
        )
        for f in range(NUM_FIELDS)
    ]
    bias_cp = pltpu.async_copy(bias_hbm, bias_v, semb)

    slice_start = sid * SLICE
    h1a = pltpu.async_copy(tmain_hbm.at[pl.ds(slice_start, HALF)],
                           tab_v.at[pl.ds(0, HALF)], semta)
    h1b = pltpu.async_copy(tmain_hbm.at[pl.ds(slice_start + HALF, HALF)],
                           tab_v.at[pl.ds(HALF, HALF)], semtb)

    @pl.when(sid == 0)
    def _issue_tail():
        pltpu.async_copy(ttail_hbm, tail_v, semt2)

    h1a.wait()
    h2a = pltpu.async_copy(tab_v.at[pl.ds(0, HALF)],
                           spt.at[pl.ds(slice_start, HALF)], semta)
    h1b.wait()
    h2b = pltpu.async_copy(tab_v.at[pl.ds(HALF, HALF)],
                           spt.at[pl.ds(slice_start + HALF, HALF)], semtb)

    @pl.when(sid == 0)
    def _bounce_tail():
        pltpu.make_async_copy(ttail_hbm, tail_v, semt2).wait()
        pltpu.sync_copy(tail_v, spt.at[pl.ds(MAIN_N, TAIL_LEN)])

    h2a.wait()
    h2b.wait()
    plsc.subcore_barrier()

    for cp in stage:
        cp.wait()

    n0 = F_G0 * ROWS_PER_W
    n1 = (NUM_FIELDS - F_G0) * ROWS_PER_W
    g0 = pltpu.async_copy(spt.at[idx_v.at[pl.ds(0, n0)]],
                          vals_v.at[pl.ds(0, n0)], semg0)
    g1 = pltpu.async_copy(spt.at[idx_v.at[pl.ds(n0, n1)]],
                          vals_v.at[pl.ds(n0, n1)], semg1)

    bias_cp.wait()
    bias_vec = plsc.load_gather(bias_v, [jnp.zeros((LANES,), jnp.int32)])

    nchunk = ROWS_PER_W // LANES
    accs = [bias_vec] * nchunk
    g0.wait()
    for j in range(nchunk):
        acc = accs[j]
        for f in range(F_G0):
            acc = acc + vals_v[pl.ds(f * ROWS_PER_W + j * LANES, LANES)]
        accs[j] = acc
    g1.wait()
    for j in range(nchunk):
        acc = accs[j]
        for f in range(F_G0, NUM_FIELDS):
            acc = acc + vals_v[pl.ds(f * ROWS_PER_W + j * LANES, LANES)]
        out_v[pl.ds(j * LANES, LANES)] = acc

    pltpu.sync_copy(out_v, out_hbm.at[pl.ds(base, ROWS_PER_W)])


@jax.jit
def _run(xt, tmain, ttail, bias):
    mesh = plsc.VectorSubcoreMesh(
        core_axis_name="c", subcore_axis_name="s",
        num_cores=NC, num_subcores=NS)
    f = functools.partial(
        pl.kernel,
        out_type=jax.ShapeDtypeStruct((BATCH,), jnp.float32),
        mesh=mesh,
        scratch_types=[
            pltpu.VMEM_SHARED((VOCAB_N,), jnp.float32),
            pltpu.VMEM((NUM_FIELDS * ROWS_PER_W,), jnp.int32),
            pltpu.VMEM((NUM_FIELDS * ROWS_PER_W,), jnp.float32),
            pltpu.VMEM((ROWS_PER_W,), jnp.float32),
            pltpu.VMEM((1,), jnp.float32),
            pltpu.VMEM((SLICE,), jnp.float32),
            pltpu.VMEM((TAIL_LEN,), jnp.float32),
            pltpu.SemaphoreType.DMA,
            pltpu.SemaphoreType.DMA,
            pltpu.SemaphoreType.DMA,
            pltpu.SemaphoreType.DMA,
            pltpu.SemaphoreType.DMA,
            pltpu.SemaphoreType.DMA,
            pltpu.SemaphoreType.DMA,
        ],
        compiler_params=pltpu.CompilerParams(needs_layout_passes=False),
    )(_sc_kernel)
    return f(xt, tmain, ttail, bias)


def kernel(x, table, bias):
    xt = x.astype(jnp.int32).T
    tmain = jax.lax.slice(table, (0, 0), (MAIN_N, 1)).reshape(MAIN_N)
    ttail = jax.lax.slice(table, (MAIN_N, 0), (VOCAB_N, 1)).reshape(TAIL_LEN)
    out = _run(xt, tmain, ttail, bias.astype(jnp.float32))
    return out.reshape(BATCH, 1)


# speedup vs baseline: 1.0366x; 1.0366x over previous
import functools

import jax
import jax.numpy as jnp
from jax import lax
from jax.experimental import pallas as pl
from jax.experimental.pallas import tpu as pltpu
from jax.experimental.pallas import tpu_sc as plsc

BATCH = 4096
NUM_FIELDS = 26
NC = 2
NS = 16
LANES = 16
NW = NC * NS
ROWS_PER_W = BATCH // NW
VOCAB_N = 100000
SLICE = 6256


def _sc_kernel(xt_hbm, table_hbm, bias_hbm, out_hbm, spt, idx_v, vals_v,
               out_v, bias_v, tab_v, sem):
    cid = lax.axis_index("c")
    sid = lax.axis_index("s")
    wid = sid * NC + cid
    base = wid * ROWS_PER_W

    def stage_body(f, _):
        pltpu.async_copy(
            xt_hbm.at[f, pl.ds(base, ROWS_PER_W)],
            idx_v.at[pl.ds(f * ROWS_PER_W, ROWS_PER_W)],
            sem,
        )
        return 0

    lax.fori_loop(0, NUM_FIELDS, stage_body, 0)
    bias_cp = pltpu.async_copy(bias_hbm, bias_v, sem)

    slice_start = sid * SLICE
    last_start = (NS - 1) * SLICE

    @pl.when(sid != NS - 1)
    def _copy_full():
        pltpu.sync_copy(table_hbm.at[pl.ds(slice_start, SLICE)], tab_v)
        pltpu.sync_copy(tab_v, spt.at[pl.ds(slice_start, SLICE)])

    @pl.when(sid == NS - 1)
    def _copy_tail():
        n = VOCAB_N - last_start
        pltpu.sync_copy(table_hbm.at[pl.ds(last_start, n)],
                        tab_v.at[pl.ds(0, n)])
        pltpu.sync_copy(tab_v.at[pl.ds(0, n)],
                        spt.at[pl.ds(last_start, n)])

    plsc.subcore_barrier()

    def drain_body(f, _):
        pltpu.make_async_copy(
            xt_hbm.at[0, pl.ds(0, ROWS_PER_W)],
            idx_v.at[pl.ds(0, ROWS_PER_W)],
            sem,
        ).wait()
        return 0

    lax.fori_loop(0, NUM_FIELDS, drain_body, 0)

    pltpu.async_copy(spt.at[idx_v], vals_v, sem).wait()

    bias_cp.wait()
    bias_vec = plsc.load_gather(bias_v, [jnp.zeros((LANES,), jnp.int32)])

    def chunk_body(j, _):
        col = j * LANES
        acc = bias_vec
        for f in range(NUM_FIELDS):
            acc = acc + vals_v[pl.ds(f * ROWS_PER_W + col, LANES)]
        out_v[pl.ds(col, LANES)] = acc
        return 0

    lax.fori_loop(0, ROWS_PER_W // LANES, chunk_body, 0)

    pltpu.sync_copy(out_v, out_hbm.at[pl.ds(base, ROWS_PER_W)])


@jax.jit
def _run(xt, table_flat, bias):
    mesh = plsc.VectorSubcoreMesh(
        core_axis_name="c", subcore_axis_name="s",
        num_cores=NC, num_subcores=NS)
    f = functools.partial(
        pl.kernel,
        out_type=jax.ShapeDtypeStruct((BATCH,), jnp.float32),
        mesh=mesh,
        scratch_types=[
            pltpu.VMEM_SHARED((NS * SLICE,), jnp.float32),
            pltpu.VMEM((NUM_FIELDS * ROWS_PER_W,), jnp.int32),
            pltpu.VMEM((NUM_FIELDS * ROWS_PER_W,), jnp.float32),
            pltpu.VMEM((ROWS_PER_W,), jnp.float32),
            pltpu.VMEM((1,), jnp.float32),
            pltpu.VMEM((SLICE,), jnp.float32),
            pltpu.SemaphoreType.DMA,
        ],
        compiler_params=pltpu.CompilerParams(needs_layout_passes=False),
    )(_sc_kernel)
    return f(xt, table_flat, bias)


def kernel(x, table, bias):
    xt = x.astype(jnp.int32).T
    table_flat = table.reshape(-1)
    out = _run(xt, table_flat, bias.astype(jnp.float32))
    return out.reshape(BATCH, 1)


# speedup vs baseline: 1.0464x; 1.0095x over previous
import functools

import jax
import jax.numpy as jnp
from jax import lax
from jax.experimental import pallas as pl
from jax.experimental.pallas import tpu as pltpu
from jax.experimental.pallas import tpu_sc as plsc

BATCH = 4096
NUM_FIELDS = 26
NC = 2
NS = 16
LANES = 16
NW = NC * NS
ROWS_PER_W = BATCH // NW
VOCAB_N = 100000
SLICE = 6256


def _sc_kernel(xt_hbm, table_hbm, bias_hbm, out_hbm, spt, idx_v, vals_v,
               out_v, bias_v, tab_v, sem, semt):
    cid = lax.axis_index("c")
    sid = lax.axis_index("s")
    wid = sid * NC + cid
    base = wid * ROWS_PER_W

    slice_start = sid * SLICE
    last_start = (NS - 1) * SLICE
    h1_full = pltpu.make_async_copy(
        table_hbm.at[pl.ds(slice_start, SLICE)], tab_v, semt)
    h1_tail = pltpu.make_async_copy(
        table_hbm.at[pl.ds(last_start, VOCAB_N - last_start)],
        tab_v.at[pl.ds(0, VOCAB_N - last_start)], semt)

    @pl.when(sid != NS - 1)
    def _issue_full():
        h1_full.start()

    @pl.when(sid == NS - 1)
    def _issue_tail():
        h1_tail.start()

    def stage_body(f, _):
        pltpu.async_copy(
            xt_hbm.at[f, pl.ds(base, ROWS_PER_W)],
            idx_v.at[pl.ds(f * ROWS_PER_W, ROWS_PER_W)],
            sem,
        )
        return 0

    lax.fori_loop(0, NUM_FIELDS, stage_body, 0)
    bias_cp = pltpu.async_copy(bias_hbm, bias_v, sem)

    @pl.when(sid != NS - 1)
    def _bounce_full():
        h1_full.wait()
        pltpu.sync_copy(tab_v, spt.at[pl.ds(slice_start, SLICE)])

    @pl.when(sid == NS - 1)
    def _bounce_tail():
        n = VOCAB_N - last_start
        h1_tail.wait()
        pltpu.sync_copy(tab_v.at[pl.ds(0, n)],
                        spt.at[pl.ds(last_start, n)])

    def drain_body(f, _):
        pltpu.make_async_copy(
            xt_hbm.at[0, pl.ds(0, ROWS_PER_W)],
            idx_v.at[pl.ds(0, ROWS_PER_W)],
            sem,
        ).wait()
        return 0

    lax.fori_loop(0, NUM_FIELDS, drain_body, 0)

    plsc.subcore_barrier()

    pltpu.async_copy(spt.at[idx_v], vals_v, sem).wait()

    bias_cp.wait()
    bias_vec = plsc.load_gather(bias_v, [jnp.zeros((LANES,), jnp.int32)])

    def chunk_body(j, _):
        col = j * LANES
        acc = bias_vec
        for f in range(NUM_FIELDS):
            acc = acc + vals_v[pl.ds(f * ROWS_PER_W + col, LANES)]
        out_v[pl.ds(col, LANES)] = acc
        return 0

    lax.fori_loop(0, ROWS_PER_W // LANES, chunk_body, 0)

    pltpu.sync_copy(out_v, out_hbm.at[pl.ds(base, ROWS_PER_W)])


@jax.jit
def _run(xt, table_flat, bias):
    mesh = plsc.VectorSubcoreMesh(
        core_axis_name="c", subcore_axis_name="s",
        num_cores=NC, num_subcores=NS)
    f = functools.partial(
        pl.kernel,
        out_type=jax.ShapeDtypeStruct((BATCH,), jnp.float32),
        mesh=mesh,
        scratch_types=[
            pltpu.VMEM_SHARED((NS * SLICE,), jnp.float32),
            pltpu.VMEM((NUM_FIELDS * ROWS_PER_W,), jnp.int32),
            pltpu.VMEM((NUM_FIELDS * ROWS_PER_W,), jnp.float32),
            pltpu.VMEM((ROWS_PER_W,), jnp.float32),
            pltpu.VMEM((1,), jnp.float32),
            pltpu.VMEM((SLICE,), jnp.float32),
            pltpu.SemaphoreType.DMA,
            pltpu.SemaphoreType.DMA,
        ],
        compiler_params=pltpu.CompilerParams(needs_layout_passes=False),
    )(_sc_kernel)
    return f(xt, table_flat, bias)


def kernel(x, table, bias):
    xt = x.astype(jnp.int32).T
    table_flat = table.reshape(-1)
    out = _run(xt, table_flat, bias.astype(jnp.float32))
    return out.reshape(BATCH, 1)


# speedup vs baseline: 1.0525x; 1.0057x over previous
import functools

import jax
import jax.numpy as jnp
from jax import lax
from jax.experimental import pallas as pl
from jax.experimental.pallas import tpu as pltpu
from jax.experimental.pallas import tpu_sc as plsc

BATCH = 4096
NUM_FIELDS = 26
NC = 2
NS = 16
LANES = 16
NW = NC * NS
ROWS_PER_W = BATCH // NW
VOCAB_N = 100000
SLICE = 6256


def _sc_kernel(xt_hbm, table_hbm, bias_hbm, out_hbm, spt, idx_v, vals_v,
               out_v, bias_v, tab_v, sem, semt):
    cid = lax.axis_index("c")
    sid = lax.axis_index("s")
    wid = sid * NC + cid
    base = wid * ROWS_PER_W

    slice_start = sid * SLICE
    last_start = (NS - 1) * SLICE
    h1_full = pltpu.make_async_copy(
        table_hbm.at[pl.ds(slice_start, SLICE)], tab_v, semt)
    h1_tail = pltpu.make_async_copy(
        table_hbm.at[pl.ds(last_start, VOCAB_N - last_start)],
        tab_v.at[pl.ds(0, VOCAB_N - last_start)], semt)

    @pl.when(sid != NS - 1)
    def _issue_full():
        h1_full.start()

    @pl.when(sid == NS - 1)
    def _issue_tail():
        h1_tail.start()

    def stage_body(f, _):
        pltpu.async_copy(
            xt_hbm.at[f, pl.ds(base, ROWS_PER_W)],
            idx_v.at[pl.ds(f * ROWS_PER_W, ROWS_PER_W)],
            sem,
        )
        return 0

    lax.fori_loop(0, NUM_FIELDS, stage_body, 0)
    bias_cp = pltpu.async_copy(bias_hbm, bias_v, sem)

    @pl.when(sid != NS - 1)
    def _bounce_full():
        h1_full.wait()
        pltpu.sync_copy(tab_v, spt.at[pl.ds(slice_start, SLICE)])

    @pl.when(sid == NS - 1)
    def _bounce_tail():
        n = VOCAB_N - last_start
        h1_tail.wait()
        pltpu.sync_copy(tab_v.at[pl.ds(0, n)],
                        spt.at[pl.ds(last_start, n)])

    pltpu.make_async_copy(
        xt_hbm.at[0, pl.ds(0, NUM_FIELDS * ROWS_PER_W)],
        idx_v,
        sem,
    ).wait()

    plsc.subcore_barrier()

    pltpu.async_copy(spt.at[idx_v], vals_v, sem).wait()

    bias_cp.wait()
    bias_vec = plsc.load_gather(bias_v, [jnp.zeros((LANES,), jnp.int32)])

    def chunk_body(j, _):
        col = j * LANES
        acc = bias_vec
        for f in range(NUM_FIELDS):
            acc = acc + vals_v[pl.ds(f * ROWS_PER_W + col, LANES)]
        out_v[pl.ds(col, LANES)] = acc
        return 0

    lax.fori_loop(0, ROWS_PER_W // LANES, chunk_body, 0)

    pltpu.sync_copy(out_v, out_hbm.at[pl.ds(base, ROWS_PER_W)])


@jax.jit
def _run(xt, table_flat, bias):
    mesh = plsc.VectorSubcoreMesh(
        core_axis_name="c", subcore_axis_name="s",
        num_cores=NC, num_subcores=NS)
    f = functools.partial(
        pl.kernel,
        out_type=jax.ShapeDtypeStruct((BATCH,), jnp.float32),
        mesh=mesh,
        scratch_types=[
            pltpu.VMEM_SHARED((NS * SLICE,), jnp.float32),
            pltpu.VMEM((NUM_FIELDS * ROWS_PER_W,), jnp.int32),
            pltpu.VMEM((NUM_FIELDS * ROWS_PER_W,), jnp.float32),
            pltpu.VMEM((ROWS_PER_W,), jnp.float32),
            pltpu.VMEM((1,), jnp.float32),
            pltpu.VMEM((SLICE,), jnp.float32),
            pltpu.SemaphoreType.DMA,
            pltpu.SemaphoreType.DMA,
        ],
        compiler_params=pltpu.CompilerParams(needs_layout_passes=False),
    )(_sc_kernel)
    return f(xt, table_flat, bias)


def kernel(x, table, bias):
    xt = x.astype(jnp.int32).T
    table_flat = table.reshape(-1)
    out = _run(xt, table_flat, bias.astype(jnp.float32))
    return out.reshape(BATCH, 1)


# speedup vs baseline: 1.0539x; 1.0013x over previous
import functools

import jax
import jax.numpy as jnp
from jax import lax
from jax.experimental import pallas as pl
from jax.experimental.pallas import tpu as pltpu
from jax.experimental.pallas import tpu_sc as plsc

BATCH = 4096
NUM_FIELDS = 26
NC = 2
NS = 16
LANES = 16
NW = NC * NS
ROWS_PER_W = BATCH // NW
VOCAB_N = 100000
SLICE = 6256


def _sc_kernel(xt_hbm, table_hbm, bias_hbm, out_hbm, spt, idx_v, vals_v,
               out_v, bias_v, tab_v, sem, semt, semb):
    cid = lax.axis_index("c")
    sid = lax.axis_index("s")
    wid = sid * NC + cid
    base = wid * ROWS_PER_W

    slice_start = sid * SLICE
    last_start = (NS - 1) * SLICE
    h1_full = pltpu.make_async_copy(
        table_hbm.at[pl.ds(slice_start, SLICE)], tab_v, semt)
    h1_tail = pltpu.make_async_copy(
        table_hbm.at[pl.ds(last_start, VOCAB_N - last_start)],
        tab_v.at[pl.ds(0, VOCAB_N - last_start)], semt)

    @pl.when(sid != NS - 1)
    def _issue_full():
        h1_full.start()

    @pl.when(sid == NS - 1)
    def _issue_tail():
        h1_tail.start()

    def stage_body(f, _):
        pltpu.async_copy(
            xt_hbm.at[f, pl.ds(base, ROWS_PER_W)],
            idx_v.at[pl.ds(f * ROWS_PER_W, ROWS_PER_W)],
            sem,
        )
        return 0

    lax.fori_loop(0, NUM_FIELDS, stage_body, 0)
    bias_cp = pltpu.async_copy(bias_hbm, bias_v, semb)

    @pl.when(sid != NS - 1)
    def _bounce_full():
        h1_full.wait()
        pltpu.sync_copy(tab_v, spt.at[pl.ds(slice_start, SLICE)])

    @pl.when(sid == NS - 1)
    def _bounce_tail():
        n = VOCAB_N - last_start
        h1_tail.wait()
        pltpu.sync_copy(tab_v.at[pl.ds(0, n)],
                        spt.at[pl.ds(last_start, n)])

    pltpu.make_async_copy(
        xt_hbm.at[0, pl.ds(0, NUM_FIELDS * ROWS_PER_W)],
        idx_v,
        sem,
    ).wait()

    plsc.subcore_barrier()

    pltpu.async_copy(spt.at[idx_v], vals_v, sem).wait()

    bias_cp.wait()
    bias_vec = plsc.load_gather(bias_v, [jnp.zeros((LANES,), jnp.int32)])

    def chunk_body(j, _):
        col = j * LANES
        acc = bias_vec
        for f in range(NUM_FIELDS):
            acc = acc + vals_v[pl.ds(f * ROWS_PER_W + col, LANES)]
        out_v[pl.ds(col, LANES)] = acc
        return 0

    lax.fori_loop(0, ROWS_PER_W // LANES, chunk_body, 0)

    pltpu.sync_copy(out_v, out_hbm.at[pl.ds(base, ROWS_PER_W)])


@jax.jit
def _run(xt, table_flat, bias):
    mesh = plsc.VectorSubcoreMesh(
        core_axis_name="c", subcore_axis_name="s",
        num_cores=NC, num_subcores=NS)
    f = functools.partial(
        pl.kernel,
        out_type=jax.ShapeDtypeStruct((BATCH,), jnp.float32),
        mesh=mesh,
        scratch_types=[
            pltpu.VMEM_SHARED((NS * SLICE,), jnp.float32),
            pltpu.VMEM((NUM_FIELDS * ROWS_PER_W,), jnp.int32),
            pltpu.VMEM((NUM_FIELDS * ROWS_PER_W,), jnp.float32),
            pltpu.VMEM((ROWS_PER_W,), jnp.float32),
            pltpu.VMEM((1,), jnp.float32),
            pltpu.VMEM((SLICE,), jnp.float32),
            pltpu.SemaphoreType.DMA,
            pltpu.SemaphoreType.DMA,
            pltpu.SemaphoreType.DMA,
        ],
        compiler_params=pltpu.CompilerParams(needs_layout_passes=False),
    )(_sc_kernel)
    return f(xt, table_flat, bias)


def kernel(x, table, bias):
    xt = x.astype(jnp.int32).T
    table_flat = table.reshape(-1)
    out = _run(xt, table_flat, bias.astype(jnp.float32))
    return out.reshape(BATCH, 1)
